# baseline (speedup 1.0000x reference)
import functools

import jax
import jax.numpy as jnp
from jax import lax
from jax.experimental import pallas as pl
from jax.experimental.pallas import tpu as pltpu
from jax.experimental.pallas import tpu_sc as plsc

_N = 10000
_D = 128
_E = 320000
_NC = 2
_NS = 16
_HN = 5120
_CH = 16
_NB = 3
_ERW = 128
_EPAD = 327680
_RPT = 88
_WIN = 1152
_IST = 8
_ACC = 5248
_ZPT = _ACC // _NS
_OPT = 624
_WPT = _HN // _NS
_EPS = 1e-5


def _seg_sum_body(h_hbm, src_hbm, dst0_hbm, dst1_hbm, out_hbm,
                  src_v, dst_v, rows0, rows1, rows2, h_sp, acc_sh,
                  sg0, sg1, sg2, ss0, ss1, ss2):
    rows = (rows0, rows1, rows2)
    sg = (sg0, sg1, sg2)
    ss = (ss0, ss1, ss2)
    c = lax.axis_index("c")
    s = lax.axis_index("s")
    tail = _N - _NS * _OPT

    @pl.loop(0, _CH)
    def _zero_fill(rr):
        for j in range(_D // 16):
            rows0[rr, pl.ds(j * 16, 16)] = jnp.zeros((16,), jnp.float32)

    @pl.loop(0, _ZPT // 8)
    def _zero_acc(k):
        pltpu.sync_copy(rows0.at[pl.ds(0, 8)],
                        acc_sh.at[pl.ds(s * _ZPT + k * 8, 8)])

    pltpu.sync_copy(h_hbm.at[pl.ds(s * _OPT, _OPT)],
                    h_sp.at[pl.ds(s * _OPT, _OPT)])

    @pl.when(s == _NS - 1)
    def _stage_tail():
        pltpu.sync_copy(h_hbm.at[pl.ds(_NS * _OPT, tail)],
                        h_sp.at[pl.ds(_NS * _OPT, tail)])

    plsc.subcore_barrier()

    for q in range(_RPT // _IST):
        base = c * _WIN + s * _RPT + q * _IST

        @pl.when(c == 0)
        def _stage_idx0():
            pltpu.sync_copy(dst0_hbm.at[pl.ds(base, _IST)], dst_v)

        @pl.when(c == 1)
        def _stage_idx1():
            pltpu.sync_copy(dst1_hbm.at[pl.ds(base, _IST)], dst_v)

        pltpu.sync_copy(src_hbm.at[pl.ds(base, _IST)], src_v)

        @pl.loop(0, _IST)
        def _row(r):
            nch = _ERW // _CH
            gs, scs = {}, {}

            def gather(j):
                si = src_v[r, pl.ds(j * _CH, _CH)]
                gs[j] = pltpu.async_copy(h_sp.at[si], rows[j % _NB],
                                         sg[j % _NB])

            for j in range(_NB):
                gather(j)
            for j in range(nch):
                b = j % _NB
                gs[j].wait()
                di = dst_v[r, pl.ds(j * _CH, _CH)]
                scs[j] = pltpu.async_copy(rows[b], acc_sh.at[di], ss[b],
                                          add=True)
                if j + _NB < nch:
                    scs[j].wait()
                    gather(j + _NB)
            for j in range(nch - _NB, nch):
                scs[j].wait()

    plsc.subcore_barrier()
    pltpu.sync_copy(acc_sh.at[pl.ds(s * _WPT, _WPT)],
                    out_hbm.at[pl.ds(c * _HN + s * _WPT, _WPT)])


@functools.lru_cache(maxsize=None)
def _get_seg_sum():
  return pl.kernel(
    _seg_sum_body,
    out_type=jax.ShapeDtypeStruct((_NC * _HN, _D), jnp.float32),
    mesh=plsc.VectorSubcoreMesh(core_axis_name="c", subcore_axis_name="s",
                                num_cores=_NC, num_subcores=_NS),
    scratch_types=(
        [pltpu.VMEM((_IST, _ERW), jnp.int32)] * 2
        + [pltpu.VMEM((_CH, _D), jnp.float32)] * _NB
        + [pltpu.VMEM_SHARED((_N, _D), jnp.float32)]
        + [pltpu.VMEM_SHARED((_ACC, _D), jnp.float32)]
        + [pltpu.SemaphoreType.DMA] * (2 * _NB)
    ),
  )


def _bn_relu(z, g, b):
    mu = jnp.mean(z, axis=0, keepdims=True)
    var = jnp.mean((z - mu) ** 2, axis=0, keepdims=True)
    return jnp.maximum((z - mu) / jnp.sqrt(var + _EPS) * g + b, 0.0)


def _mlp_tower(h, agg, w1, b1, g1, be1, w2, b2, g2, be2, gg, gb):
    z = h + agg
    z = _bn_relu(jnp.dot(z, w1[...], preferred_element_type=jnp.float32)
                 + b1[...], g1[...], be1[...])
    z = _bn_relu(jnp.dot(z, w2[...], preferred_element_type=jnp.float32)
                 + b2[...], g2[...], be2[...])
    return _bn_relu(z, gg[...], gb[...])


def _layer_first_body(h_ref, agg_ref, w1, b1, g1, be1, w2, b2, g2, be2,
                      gg, gb, paw, pab, pbw, pbb, out_h, out_sc):
    h = h_ref[...]
    agg = agg_ref[...]
    z = _mlp_tower(h, agg[:_N], w1, b1, g1, be1, w2, b2, g2, be2, gg, gb)
    out_h[...] = z
    sc = jnp.dot(jnp.sum(h, 0, keepdims=True), paw[...],
                 preferred_element_type=jnp.float32) + pab[...]
    sc = sc + jnp.dot(jnp.sum(z, 0, keepdims=True), pbw[...],
                      preferred_element_type=jnp.float32) + pbb[...]
    out_sc[...] = sc


def _layer_rest_body(h_ref, agg_ref, w1, b1, g1, be1, w2, b2, g2, be2,
                     gg, gb, pbw, pbb, sin, out_h, out_sc):
    h = h_ref[...]
    agg = agg_ref[...]
    z = _mlp_tower(h, agg[:_N], w1, b1, g1, be1, w2, b2, g2, be2, gg, gb)
    out_h[...] = z
    sc = sin[...] + jnp.dot(jnp.sum(z, 0, keepdims=True), pbw[...],
                            preferred_element_type=jnp.float32) + pbb[...]
    out_sc[...] = sc


_layer_out = [jax.ShapeDtypeStruct((_N, _D), jnp.float32),
              jax.ShapeDtypeStruct((1, _D), jnp.float32)]

_layer_first = pl.pallas_call(_layer_first_body, out_shape=_layer_out)
_layer_rest = pl.pallas_call(_layer_rest_body, out_shape=_layer_out)


def _run_layers(x, src_p, dst0_p, dst1_p, params):
    r = lambda v: v.reshape(1, _D)
    h = x
    score = None
    for l in range(4):
        agg = _get_seg_sum()(h, src_p, dst0_p, dst1_p)
        p = params['gin'][l]
        pn = params['pred'][l + 1]
        common = (h, agg, p['W1'], r(p['b1']), r(p['g1']), r(p['be1']),
                  p['W2'], r(p['b2']), r(p['g2']), r(p['be2']),
                  r(p['gbn_g']), r(p['gbn_b']))
        if l == 0:
            p0 = params['pred'][0]
            h, score = _layer_first(*common, p0['W'], r(p0['b']),
                                    pn['W'], r(pn['b']))
        else:
            h, score = _layer_rest(*common, pn['W'], r(pn['b']), score)
    return score


def kernel(x, edge_index, params):
    src = edge_index[0]
    dst = edge_index[1]
    pad = _EPAD - _E
    src_p = jnp.concatenate([src, jnp.zeros((pad,), jnp.int32)])
    dst_p = jnp.concatenate([dst, jnp.full((pad,), _N, jnp.int32)])
    in_a = dst_p < _HN
    pos_a = jnp.cumsum(in_a.astype(jnp.int32)) - 1
    pos_b = jnp.cumsum(1 - in_a.astype(jnp.int32)) - 1
    n_a = pos_a[-1] + 1
    slot = jnp.where(in_a, pos_a, n_a + pos_b)
    packed = (src_p << 15) | dst_p
    packed = jnp.zeros((_EPAD,), jnp.int32).at[slot].set(
        packed, unique_indices=True)
    src_p = packed >> 15
    dst_p = packed & 0x7FFF
    dummy = _HN + (dst_p & 127)
    rows = _EPAD // _ERW
    src_p = src_p.reshape(rows, _ERW)
    dst0 = jnp.where(dst_p < _HN, dst_p, dummy).reshape(rows, _ERW)
    dst1 = jnp.where(dst_p >= _HN, dst_p - _HN, dummy).reshape(rows, _ERW)
    return _run_layers(x, src_p, dst0, dst1, params)


# speedup vs baseline: 1.6986x; 1.6986x over previous
import functools

import jax
import jax.numpy as jnp
from jax import lax
from jax.experimental import pallas as pl
from jax.experimental.pallas import tpu as pltpu
from jax.experimental.pallas import tpu_sc as plsc

_N = 10000
_D = 128
_E = 320000
_NC = 2
_NS = 16
_HN = 5120
_CH = 16
_NB = 3
_ERW = 128
_EPAD = 327680
_RPT = 88
_WIN = 1152
_IST = 8
_ACC = 5248
_ZPT = _ACC // _NS
_OPT = 624
_WPT = _HN // _NS
_EPS = 1e-5


def _seg_sum_body(h_hbm, src_hbm, dst0_hbm, dst1_hbm, out_hbm,
                  src_v, dst_v, rows0, rows1, rows2, h_sp, acc_sh,
                  sg0, sg1, sg2, ss0, ss1, ss2):
    rows = (rows0, rows1, rows2)
    sg = (sg0, sg1, sg2)
    ss = (ss0, ss1, ss2)
    c = lax.axis_index("c")
    s = lax.axis_index("s")
    tail = _N - _NS * _OPT

    @pl.loop(0, _CH)
    def _zero_fill(rr):
        for j in range(_D // 16):
            rows0[rr, pl.ds(j * 16, 16)] = jnp.zeros((16,), jnp.float32)

    @pl.loop(0, _ZPT // 8)
    def _zero_acc(k):
        pltpu.sync_copy(rows0.at[pl.ds(0, 8)],
                        acc_sh.at[pl.ds(s * _ZPT + k * 8, 8)])

    pltpu.sync_copy(h_hbm.at[pl.ds(s * _OPT, _OPT)],
                    h_sp.at[pl.ds(s * _OPT, _OPT)])

    @pl.when(s == _NS - 1)
    def _stage_tail():
        pltpu.sync_copy(h_hbm.at[pl.ds(_NS * _OPT, tail)],
                        h_sp.at[pl.ds(_NS * _OPT, tail)])

    plsc.subcore_barrier()

    for q in range(_RPT // _IST):
        base = c * _WIN + s * _RPT + q * _IST

        @pl.when(c == 0)
        def _stage_idx0():
            pltpu.sync_copy(dst0_hbm.at[pl.ds(base, _IST)], dst_v)

        @pl.when(c == 1)
        def _stage_idx1():
            pltpu.sync_copy(dst1_hbm.at[pl.ds(base, _IST)], dst_v)

        pltpu.sync_copy(src_hbm.at[pl.ds(base, _IST)], src_v)

        @pl.loop(0, _IST)
        def _row(r):
            nch = _ERW // _CH
            gs, scs = {}, {}

            def gather(j):
                si = src_v[r, pl.ds(j * _CH, _CH)]
                gs[j] = pltpu.async_copy(h_sp.at[si], rows[j % _NB],
                                         sg[j % _NB])

            for j in range(_NB):
                gather(j)
            for j in range(nch):
                b = j % _NB
                gs[j].wait()
                di = dst_v[r, pl.ds(j * _CH, _CH)]
                scs[j] = pltpu.async_copy(rows[b], acc_sh.at[di], ss[b],
                                          add=True)
                if j + _NB < nch:
                    scs[j].wait()
                    gather(j + _NB)
            for j in range(nch - _NB, nch):
                scs[j].wait()

    plsc.subcore_barrier()
    pltpu.sync_copy(acc_sh.at[pl.ds(s * _WPT, _WPT)],
                    out_hbm.at[pl.ds(c * _HN + s * _WPT, _WPT)])


@functools.lru_cache(maxsize=None)
def _get_seg_sum():
  return pl.kernel(
    _seg_sum_body,
    out_type=jax.ShapeDtypeStruct((_NC * _HN, _D), jnp.float32),
    mesh=plsc.VectorSubcoreMesh(core_axis_name="c", subcore_axis_name="s",
                                num_cores=_NC, num_subcores=_NS),
    scratch_types=(
        [pltpu.VMEM((_IST, _ERW), jnp.int32)] * 2
        + [pltpu.VMEM((_CH, _D), jnp.float32)] * _NB
        + [pltpu.VMEM_SHARED((_N, _D), jnp.float32)]
        + [pltpu.VMEM_SHARED((_ACC, _D), jnp.float32)]
        + [pltpu.SemaphoreType.DMA] * (2 * _NB)
    ),
  )


def _bn_relu(z, g, b):
    mu = jnp.mean(z, axis=0, keepdims=True)
    var = jnp.mean((z - mu) ** 2, axis=0, keepdims=True)
    return jnp.maximum((z - mu) / jnp.sqrt(var + _EPS) * g + b, 0.0)


def _mlp_tower(h, agg, w1, b1, g1, be1, w2, b2, g2, be2, gg, gb):
    z = h + agg
    z = _bn_relu(jnp.dot(z, w1[...], preferred_element_type=jnp.float32)
                 + b1[...], g1[...], be1[...])
    z = _bn_relu(jnp.dot(z, w2[...], preferred_element_type=jnp.float32)
                 + b2[...], g2[...], be2[...])
    return _bn_relu(z, gg[...], gb[...])


def _layer_first_body(h_ref, agg_ref, w1, b1, g1, be1, w2, b2, g2, be2,
                      gg, gb, paw, pab, pbw, pbb, out_h, out_sc):
    h = h_ref[...]
    agg = agg_ref[...]
    z = _mlp_tower(h, agg[:_N], w1, b1, g1, be1, w2, b2, g2, be2, gg, gb)
    out_h[...] = z
    sc = jnp.dot(jnp.sum(h, 0, keepdims=True), paw[...],
                 preferred_element_type=jnp.float32) + pab[...]
    sc = sc + jnp.dot(jnp.sum(z, 0, keepdims=True), pbw[...],
                      preferred_element_type=jnp.float32) + pbb[...]
    out_sc[...] = sc


def _layer_rest_body(h_ref, agg_ref, w1, b1, g1, be1, w2, b2, g2, be2,
                     gg, gb, pbw, pbb, sin, out_h, out_sc):
    h = h_ref[...]
    agg = agg_ref[...]
    z = _mlp_tower(h, agg[:_N], w1, b1, g1, be1, w2, b2, g2, be2, gg, gb)
    out_h[...] = z
    sc = sin[...] + jnp.dot(jnp.sum(z, 0, keepdims=True), pbw[...],
                            preferred_element_type=jnp.float32) + pbb[...]
    out_sc[...] = sc


_layer_out = [jax.ShapeDtypeStruct((_N, _D), jnp.float32),
              jax.ShapeDtypeStruct((1, _D), jnp.float32)]

_layer_first = pl.pallas_call(_layer_first_body, out_shape=_layer_out)
_layer_rest = pl.pallas_call(_layer_rest_body, out_shape=_layer_out)


def _run_layers(x, src_p, dst0_p, dst1_p, params):
    r = lambda v: v.reshape(1, _D)
    h = x
    score = None
    for l in range(4):
        agg = _get_seg_sum()(h, src_p, dst0_p, dst1_p)
        p = params['gin'][l]
        pn = params['pred'][l + 1]
        common = (h, agg, p['W1'], r(p['b1']), r(p['g1']), r(p['be1']),
                  p['W2'], r(p['b2']), r(p['g2']), r(p['be2']),
                  r(p['gbn_g']), r(p['gbn_b']))
        if l == 0:
            p0 = params['pred'][0]
            h, score = _layer_first(*common, p0['W'], r(p0['b']),
                                    pn['W'], r(pn['b']))
        else:
            h, score = _layer_rest(*common, pn['W'], r(pn['b']), score)
    return score


def kernel(x, edge_index, params):
    src = edge_index[0]
    dst = edge_index[1]
    pad = _EPAD - _E
    src_p = jnp.concatenate([src, jnp.zeros((pad,), jnp.int32)])
    dst_p = jnp.concatenate([dst, jnp.full((pad,), _N, jnp.int32)])
    order = jnp.argsort((dst_p >= _HN).astype(jnp.int32), stable=True)
    src_p = src_p[order]
    dst_p = dst_p[order]
    dummy = _HN + (dst_p & 127)
    rows = _EPAD // _ERW
    src_p = src_p.reshape(rows, _ERW)
    dst0 = jnp.where(dst_p < _HN, dst_p, dummy).reshape(rows, _ERW)
    dst1 = jnp.where(dst_p >= _HN, dst_p - _HN, dummy).reshape(rows, _ERW)
    return _run_layers(x, src_p, dst0, dst1, params)
